# Initial kernel scaffold; baseline (speedup 1.0000x reference)
import functools

import jax
import jax.numpy as jnp
from jax import lax
from jax.experimental import pallas as pl
from jax.experimental.pallas import tpu as pltpu
from jax.experimental.pallas import tpu_sc as plsc

DIGIT_DIM = 16
POS_DIM = 8
P = 81
F = 32
NW = 32
NB = 16
ROW_W = P * F


@functools.lru_cache(maxsize=None)
def _make_encoder(B: int):
    assert B % (NW * NB) == 0
    rows_per_w = B // NW
    n_chunks = rows_per_w // NB
    x_per_chunk = NB * P
    out_per_chunk = NB * ROW_W
    x_per_w = rows_per_w * P

    mesh = plsc.VectorSubcoreMesh(core_axis_name="c", subcore_axis_name="s")

    @functools.partial(
        pl.kernel,
        mesh=mesh,
        out_type=jax.ShapeDtypeStruct((B * ROW_W,), jnp.float32),
        scratch_types=[
            pltpu.VMEM((10, DIGIT_DIM), jnp.float32),
            pltpu.VMEM((144,), jnp.float32),
            pltpu.VMEM((P * 16,), jnp.float32),
            pltpu.VMEM((2, x_per_chunk), jnp.int32),
            pltpu.VMEM((2, out_per_chunk), jnp.float32),
        ],
    )
    def enc(x_hbm, digit_hbm, row_hbm, col_hbm, out_hbm,
            digit_v, rc_v, pos_v, x_v, out_v):
        wid = lax.axis_index("s") * 2 + lax.axis_index("c")
        lane = lax.iota(jnp.int32, 16)
        zeros = lane * 0

        pltpu.sync_copy(digit_hbm, digit_v)
        pltpu.sync_copy(row_hbm, rc_v.at[pl.ds(0, 72)])
        pltpu.sync_copy(col_hbm, rc_v.at[pl.ds(72, 72)])

        for p in range(P):
            r, c = p // 9, p % 9
            idx = jnp.where(lane < 8, r * 8 + lane, 64 + c * 8 + lane)
            pos_v[pl.ds(p * 16, 16)] = plsc.load_gather(rc_v, [idx])

        def fill_row(b, carry):
            for p in range(P):
                v = pos_v[pl.ds(p * 16, 16)]
                for k in (0, 1):
                    out_v[k, pl.ds(b * ROW_W + p * F + DIGIT_DIM, 16)] = v
            return carry
        lax.fori_loop(0, NB, fill_row, 0)

        def do_chunk(k, chunk):
            xbase = wid * x_per_w + chunk * x_per_chunk
            pltpu.sync_copy(x_hbm.at[pl.ds(xbase, x_per_chunk)], x_v.at[k])

            def grp(g, carry):
                xv = x_v[k, pl.ds(g * 16, 16)]
                base_o = g * 512
                for j in range(16):
                    bj = jnp.take(xv, zeros + j, mode="promise_in_bounds")
                    dv = plsc.load_gather(digit_v, [bj, lane])
                    out_v[k, pl.ds(base_o + j * F, 16)] = dv
                return carry
            lax.fori_loop(0, P, grp, 0)
            pltpu.sync_copy(out_v.at[k],
                            out_hbm.at[pl.ds(xbase * F, out_per_chunk)])

        def chunk_pair(ci, carry):
            for k in (0, 1):
                do_chunk(k, ci * 2 + k)
            return carry
        lax.fori_loop(0, n_chunks // 2, chunk_pair, 0)

    return enc


def kernel(x, digit_emb, row_emb, col_emb):
    B, p = x.shape
    assert p == P
    xf = x.reshape(-1).astype(jnp.int32)
    out = _make_encoder(B)(xf, digit_emb,
                           row_emb.reshape(-1), col_emb.reshape(-1))
    return out.reshape(B, P, F)


# speedup vs baseline: 7.2003x; 7.2003x over previous
import functools

import jax
import jax.numpy as jnp
from jax import lax
from jax.experimental import pallas as pl
from jax.experimental.pallas import tpu as pltpu
from jax.experimental.pallas import tpu_sc as plsc

DIGIT_DIM = 16
POS_DIM = 8
P = 81
F = 32
NW = 32
NB = 16
ROW_W = P * F


@functools.lru_cache(maxsize=None)
def _make_encoder(B: int):
    assert B % (NW * NB) == 0
    rows_per_w = B // NW
    n_chunks = rows_per_w // NB
    x_per_chunk = NB * P
    out_per_chunk = NB * ROW_W
    x_per_w = rows_per_w * P

    mesh = plsc.VectorSubcoreMesh(core_axis_name="c", subcore_axis_name="s")

    @functools.partial(
        pl.kernel,
        mesh=mesh,
        out_type=jax.ShapeDtypeStruct((B * ROW_W,), jnp.float32),
        scratch_types=[
            pltpu.VMEM((160,), jnp.float32),
            pltpu.VMEM((80,), jnp.float32),
            pltpu.VMEM((80,), jnp.float32),
            pltpu.VMEM((9 * 16,), jnp.float32),
            pltpu.VMEM((P * 16,), jnp.float32),
            pltpu.VMEM((x_per_chunk,), jnp.int32),
            pltpu.VMEM((x_per_chunk,), jnp.int32),
            pltpu.VMEM((2, out_per_chunk), jnp.float32),
        ],
    )
    def enc(x_hbm, digit_hbm, row_hbm, col_hbm, out_hbm,
            digit_v, row_v, col_v, colhi_v, pos_v, x0_v, x1_v, out_v):
        wid = lax.axis_index("s") * 2 + lax.axis_index("c")
        lane = lax.iota(jnp.int32, 16)

        pltpu.sync_copy(digit_hbm, digit_v.at[pl.ds(0, 160)])
        pltpu.sync_copy(row_hbm, row_v.at[pl.ds(0, 72)])
        pltpu.sync_copy(col_hbm, col_v.at[pl.ds(0, 72)])

        for c in range(9):
            cv = col_v[pl.ds(c * 8, 16)]
            acc = cv * 0.0
            for i in range(8):
                acc = jnp.where(lane == 8 + i, cv[i], acc)
            colhi_v[pl.ds(c * 16, 16)] = acc

        for p in range(P):
            r, c = p // 9, p % 9
            a = row_v[pl.ds(r * 8, 16)]
            b = colhi_v[pl.ds(c * 16, 16)]
            pos_v[pl.ds(p * 16, 16)] = jnp.where(lane < 8, a, b)

        def fill_row(b, carry):
            for p in range(P):
                v = pos_v[pl.ds(p * 16, 16)]
                for k in (0, 1):
                    out_v[k, pl.ds(b * ROW_W + p * F + DIGIT_DIM, 16)] = v
            return carry
        lax.fori_loop(0, NB, fill_row, 0)

        def do_chunk(k, xk_v, chunk):
            xbase = wid * x_per_w + chunk * x_per_chunk
            pltpu.sync_copy(x_hbm.at[pl.ds(xbase, x_per_chunk)],
                            xk_v.at[pl.ds(0, x_per_chunk)])

            def grp(g, carry):
                xv = xk_v[pl.ds(g * 16, 16)] * DIGIT_DIM
                base_o = g * 512
                for j in range(16):
                    dv = digit_v[pl.ds(xv[j], DIGIT_DIM)]
                    out_v[k, pl.ds(base_o + j * F, 16)] = dv
                return carry
            lax.fori_loop(0, P, grp, 0)
            pltpu.sync_copy(out_v.at[k],
                            out_hbm.at[pl.ds(xbase * F, out_per_chunk)])

        def chunk_pair(ci, carry):
            do_chunk(0, x0_v, ci * 2)
            do_chunk(1, x1_v, ci * 2 + 1)
            return carry
        lax.fori_loop(0, n_chunks // 2, chunk_pair, 0)

    return enc


def kernel(x, digit_emb, row_emb, col_emb):
    B, p = x.shape
    assert p == P
    xf = x.reshape(-1).astype(jnp.int32)
    out = _make_encoder(B)(xf, digit_emb.reshape(-1),
                           row_emb.reshape(-1), col_emb.reshape(-1))
    return out.reshape(B, P, F)


# speedup vs baseline: 8.8617x; 1.2307x over previous
import functools

import jax
import jax.numpy as jnp
from jax import lax
from jax.experimental import pallas as pl
from jax.experimental.pallas import tpu as pltpu
from jax.experimental.pallas import tpu_sc as plsc

DIGIT_DIM = 16
POS_DIM = 8
P = 81
F = 32
NW = 32
NB = 16
ROW_W = P * F


@functools.lru_cache(maxsize=None)
def _make_encoder(B: int):
    assert B % (NW * NB) == 0
    rows_per_w = B // NW
    n_chunks = rows_per_w // NB
    x_per_chunk = NB * P
    out_per_chunk = NB * ROW_W
    x_per_w = rows_per_w * P

    mesh = plsc.VectorSubcoreMesh(core_axis_name="c", subcore_axis_name="s")

    @functools.partial(
        pl.kernel,
        mesh=mesh,
        out_type=jax.ShapeDtypeStruct((B * ROW_W,), jnp.float32),
        scratch_types=[
            pltpu.VMEM((160,), jnp.float32),
            pltpu.VMEM((80,), jnp.float32),
            pltpu.VMEM((80,), jnp.float32),
            pltpu.VMEM((9 * 16,), jnp.float32),
            pltpu.VMEM((P * 16,), jnp.float32),
            pltpu.VMEM((x_per_chunk,), jnp.int32),
            pltpu.VMEM((x_per_chunk,), jnp.int32),
            pltpu.VMEM((2, out_per_chunk), jnp.float32),
            pltpu.SemaphoreType.DMA,
            pltpu.SemaphoreType.DMA,
            pltpu.SemaphoreType.DMA,
            pltpu.SemaphoreType.DMA,
        ],
    )
    def enc(x_hbm, digit_hbm, row_hbm, col_hbm, out_hbm,
            digit_v, row_v, col_v, colhi_v, pos_v, x0_v, x1_v, out_v,
            xsem0, xsem1, osem0, osem1):
        wid = lax.axis_index("s") * 2 + lax.axis_index("c")
        lane = lax.iota(jnp.int32, 16)

        def x_slice(chunk):
            return x_hbm.at[pl.ds(wid * x_per_w + chunk * x_per_chunk,
                                  x_per_chunk)]

        def out_slice(chunk):
            return out_hbm.at[pl.ds((wid * x_per_w + chunk * x_per_chunk) * F,
                                    out_per_chunk)]

        pltpu.async_copy(x_slice(0), x0_v.at[pl.ds(0, x_per_chunk)], xsem0)
        pltpu.async_copy(x_slice(1), x1_v.at[pl.ds(0, x_per_chunk)], xsem1)

        pltpu.sync_copy(digit_hbm, digit_v.at[pl.ds(0, 160)])
        pltpu.sync_copy(row_hbm, row_v.at[pl.ds(0, 72)])
        pltpu.sync_copy(col_hbm, col_v.at[pl.ds(0, 72)])

        for c in range(9):
            cv = col_v[pl.ds(c * 8, 16)]
            acc = cv * 0.0
            for i in range(8):
                acc = jnp.where(lane == 8 + i, cv[i], acc)
            colhi_v[pl.ds(c * 16, 16)] = acc

        for p in range(P):
            r, c = p // 9, p % 9
            a = row_v[pl.ds(r * 8, 16)]
            b = colhi_v[pl.ds(c * 16, 16)]
            pos_v[pl.ds(p * 16, 16)] = jnp.where(lane < 8, a, b)

        @plsc.parallel_loop(0, NB)
        def fill_row(b):
            for p in range(P):
                v = pos_v[pl.ds(p * 16, 16)]
                for k in (0, 1):
                    out_v[k, pl.ds(b * ROW_W + p * F + DIGIT_DIM, 16)] = v

        def do_chunk(k, xk_v, xsem, osem, chunk):
            @pl.when(chunk >= 2)
            def _drain_out():
                pltpu.make_async_copy(out_v.at[k], out_slice(chunk), osem
                                      ).wait()
            pltpu.make_async_copy(x_slice(chunk),
                                  xk_v.at[pl.ds(0, x_per_chunk)], xsem).wait()

            @plsc.parallel_loop(0, P, unroll=2)
            def grp(g):
                xv = xk_v[pl.ds(g * 16, 16)] * DIGIT_DIM
                base_o = g * 512
                for j in range(16):
                    dv = digit_v[pl.ds(xv[j], DIGIT_DIM)]
                    out_v[k, pl.ds(base_o + j * F, 16)] = dv

            pltpu.async_copy(out_v.at[k], out_slice(chunk), osem)

            @pl.when(chunk + 2 < n_chunks)
            def _prefetch_x():
                pltpu.async_copy(x_slice(chunk + 2),
                                 xk_v.at[pl.ds(0, x_per_chunk)], xsem)

        def chunk_pair(ci, carry):
            do_chunk(0, x0_v, xsem0, osem0, ci * 2)
            do_chunk(1, x1_v, xsem1, osem1, ci * 2 + 1)
            return carry
        lax.fori_loop(0, n_chunks // 2, chunk_pair, 0)

        pltpu.make_async_copy(out_v.at[0], out_slice(n_chunks - 2), osem0
                              ).wait()
        pltpu.make_async_copy(out_v.at[1], out_slice(n_chunks - 1), osem1
                              ).wait()

    return enc


def kernel(x, digit_emb, row_emb, col_emb):
    B, p = x.shape
    assert p == P
    xf = x.reshape(-1).astype(jnp.int32)
    out = _make_encoder(B)(xf, digit_emb.reshape(-1),
                           row_emb.reshape(-1), col_emb.reshape(-1))
    return out.reshape(B, P, F)
